# baseline (speedup 1.0000x reference)
import functools
import jax
import jax.numpy as jnp
from jax import lax
from jax.experimental import pallas as pl
from jax.experimental.pallas import tpu as pltpu, tpu_sc as plsc

_BG_PID = 5532.0
_BATCH = 128
_NUM_FG = 32
_FG_THRESH = 0.5
_BG_HI = 0.5
_BG_LO = 0.1
_N_PAD = 20480
_CHUNK = 2048
_NCHUNK = _N_PAD // _CHUNK
_STDS = (0.1, 0.1, 0.2, 0.2)

_NW = 16
_STRIP = _N_PAD // _NW
_NV = _STRIP // 16

_f32 = jnp.float32
_i32 = jnp.int32


def _iou_kern(roisT_ref, gt_ref, maxs_ref, amax_ref):
    qx1 = gt_ref[:, 0:1]
    qy1 = gt_ref[:, 1:2]
    qx2 = gt_ref[:, 2:3]
    qy2 = gt_ref[:, 3:4]
    qarea = (qx2 - qx1 + 1.0) * (qy2 - qy1 + 1.0)

    for c in range(_NCHUNK):
        sl = pl.ds(c * _CHUNK, _CHUNK)
        bx1 = roisT_ref[1:2, sl]
        by1 = roisT_ref[2:3, sl]
        bx2 = roisT_ref[3:4, sl]
        by2 = roisT_ref[4:5, sl]
        barea = (bx2 - bx1 + 1.0) * (by2 - by1 + 1.0)
        iw = jnp.maximum(jnp.minimum(bx2, qx2) - jnp.maximum(bx1, qx1) + 1.0, 0.0)
        ih = jnp.maximum(jnp.minimum(by2, qy2) - jnp.maximum(by1, qy1) + 1.0, 0.0)
        inter = iw * ih
        ua = barea + qarea - inter
        ov = inter / ua
        maxs_ref[c:c + 1, :] = jnp.max(ov, axis=0, keepdims=True)
        amax_ref[c:c + 1, :] = jnp.argmax(ov, axis=0, keepdims=True).astype(_f32)


def _sc_body(maxs_hbm, rois16_hbm, gt8_hbm, kept_hbm, asg_hbm,
             mx_v, keep_v, stage_v, scan_v, cnts_v, keeps_l, idx2,
             rows_v, asg_v, cnts_sh, keeps_sh, sem):
    wid = lax.axis_index("s")
    iot = lax.broadcasted_iota(_i32, (16,), 0)
    zero16 = jnp.zeros((16,), _i32)

    def prefix16(x):
        p = x
        for k in (1, 2, 4, 8):
            scan_v[...] = p
            sh = plsc.load_gather(scan_v, [jnp.maximum(iot - k, 0)])
            p = p + jnp.where(iot >= k, sh, 0)
        return p

    def tot(x):
        scan_v[...] = prefix16(x)
        return plsc.load_gather(scan_v, [jnp.full((16,), 15, _i32)])

    pltpu.sync_copy(maxs_hbm.at[pl.ds(wid * _STRIP, _STRIP)], mx_v)

    def p1(i, carry):
        cfg, cbg = carry
        v = mx_v[pl.ds(i * 16, 16)]
        fg = v >= _FG_THRESH
        bg = jnp.logical_and(v >= _BG_LO, v < _BG_HI)
        return (cfg + tot(jnp.where(fg, 1, 0)),
                cbg + tot(jnp.where(bg, 1, 0)))

    cfg, cbg = lax.fori_loop(0, _NV, p1, (zero16, zero16))

    stage_v[...] = jnp.where(iot == 0, cfg, jnp.where(iot == 1, cbg, 0))
    pltpu.sync_copy(stage_v, cnts_sh.at[wid])
    plsc.subcore_barrier()
    pltpu.sync_copy(cnts_sh, cnts_v)

    c0f = plsc.load_gather(cnts_v, [iot, zero16])
    c0b = plsc.load_gather(cnts_v, [iot, jnp.ones((16,), _i32)])
    pref_fg = tot(jnp.where(iot < wid, c0f, 0))
    pref_bg = tot(jnp.where(iot < wid, c0b, 0))

    for j in range(8):
        keep_v[pl.ds(j * 16, 16)] = zero16

    def p3(i, carry):
        rfg, rbg = carry
        v = mx_v[pl.ds(i * 16, 16)]
        fg = v >= _FG_THRESH
        bg = jnp.logical_and(v >= _BG_LO, v < _BG_HI)
        pfg = prefix16(jnp.where(fg, 1, 0))
        pbg = prefix16(jnp.where(bg, 1, 0))
        t_fg = pref_fg + rfg + pfg - 1
        t_bg = pref_bg + rbg + pbg - 1
        selfg = jnp.logical_and(fg, t_fg < _NUM_FG)
        selbg = jnp.logical_and(bg, t_bg < _BATCH - _NUM_FG)
        slot = jnp.where(selfg, t_fg, jnp.where(selbg, t_bg + _NUM_FG, 0))
        msk = jnp.logical_or(selfg, selbg)
        gidx = wid * _STRIP + i * 16 + iot
        plsc.store_scatter(keep_v, [slot], gidx, mask=msk)
        scan_v[...] = pfg
        tfg = plsc.load_gather(scan_v, [jnp.full((16,), 15, _i32)])
        scan_v[...] = pbg
        tbg = plsc.load_gather(scan_v, [jnp.full((16,), 15, _i32)])
        return rfg + tfg, rbg + tbg

    lax.fori_loop(0, _NV, p3, (zero16, zero16))

    pltpu.sync_copy(keep_v, keeps_sh.at[wid])
    plsc.subcore_barrier()

    @pl.when(wid == 0)
    def _():
        pltpu.sync_copy(keeps_sh, keeps_l)
        for j in range(8):
            acc = zero16
            for r in range(_NW):
                acc = acc + keeps_l[r, pl.ds(j * 16, 16)]
            keep_v[pl.ds(j * 16, 16)] = acc
        pltpu.async_copy(rois16_hbm.at[keep_v], rows_v, sem).wait()
        for j in range(8):
            ak = plsc.load_gather(rows_v, [j * 16 + iot, jnp.full((16,), 8, _i32)])
            idx2[pl.ds(j * 16, 16)] = ak.astype(_i32)
        pltpu.async_copy(gt8_hbm.at[idx2], asg_v, sem).wait()
        pltpu.sync_copy(rows_v, kept_hbm)
        pltpu.sync_copy(asg_v, asg_hbm)


_sc_call = functools.partial(
    pl.kernel, _sc_body,
    out_type=(jax.ShapeDtypeStruct((_BATCH, 16), _f32),
              jax.ShapeDtypeStruct((_BATCH, 8), _f32)),
    mesh=plsc.VectorSubcoreMesh(core_axis_name="c", subcore_axis_name="s",
                                num_cores=1),
    compiler_params=pltpu.CompilerParams(needs_layout_passes=False,
                                         use_tc_tiling_on_sc=False),
    scratch_types=[
        pltpu.VMEM((_STRIP,), _f32),
        pltpu.VMEM((_BATCH,), _i32),
        pltpu.VMEM((16,), _i32),
        pltpu.VMEM((16,), _i32),
        pltpu.VMEM((_NW, 16), _i32),
        pltpu.VMEM((_NW, _BATCH), _i32),
        pltpu.VMEM((_BATCH,), _i32),
        pltpu.VMEM((_BATCH, 16), _f32),
        pltpu.VMEM((_BATCH, 8), _f32),
        pltpu.VMEM_SHARED((_NW, 16), _i32),
        pltpu.VMEM_SHARED((_NW, _BATCH), _i32),
        pltpu.SemaphoreType.DMA,
    ],
)


def _final_kern(kept_ref, asg_ref,
                rois_out, lab_out, pid_out, bt_out, biw_out, bow_out):
    kept = kept_ref[...]
    assigned = asg_ref[...]
    rx1 = kept[:, 1:2]
    ry1 = kept[:, 2:3]
    rx2 = kept[:, 3:4]
    ry2 = kept[:, 4:5]
    gx1 = assigned[:, 0:1]
    gy1 = assigned[:, 1:2]
    gx2 = assigned[:, 2:3]
    gy2 = assigned[:, 3:4]
    ew = rx2 - rx1 + 1.0
    eh = ry2 - ry1 + 1.0
    ecx = rx1 + 0.5 * ew
    ecy = ry1 + 0.5 * eh
    gw = gx2 - gx1 + 1.0
    gh = gy2 - gy1 + 1.0
    gcx = gx1 + 0.5 * gw
    gcy = gy1 + 0.5 * gh
    dx = (gcx - ecx) / ew / _STDS[0]
    dy = (gcy - ecy) / eh / _STDS[1]
    dw = jnp.log(gw / ew) / _STDS[2]
    dh = jnp.log(gh / eh) / _STDS[3]
    data = jnp.concatenate([dx, dy, dw, dh], axis=1)

    srow = lax.broadcasted_iota(_i32, (_BATCH, 1), 0).astype(_f32)
    isfg_slot = srow < float(_NUM_FG)
    labels = jnp.where(isfg_slot, assigned[:, 4:5], 0.0)
    pids = jnp.where(isfg_slot, assigned[:, 5:6], _BG_PID)
    fgw = (labels > 0.0).astype(_f32)
    clsr = jnp.round(labels)
    m0 = (clsr == 0.0).astype(_f32)
    m1 = (clsr == 1.0).astype(_f32)
    d = data * fgw
    ones4 = jnp.ones((_BATCH, 4), dtype=_f32)
    bt_out[...] = jnp.concatenate([d * m0, d * m1], axis=1)
    biw_out[...] = jnp.concatenate([ones4 * (fgw * m0), ones4 * (fgw * m1)], axis=1)
    bow_out[...] = jnp.concatenate([ones4 * (fgw * m0), ones4 * (fgw * m1)], axis=1)
    rois_out[...] = kept[:, 0:5]
    lab_out[...] = labels.astype(jnp.int32)
    pid_out[...] = pids.astype(jnp.int32)


@jax.jit
def kernel(all_rois, gt_boxes):
    G = gt_boxes.shape[0]
    gt_rois = jnp.concatenate(
        [jnp.zeros((G, 1), jnp.float32), gt_boxes[:, :4]], axis=1)
    rois_ext = jnp.concatenate([all_rois, gt_rois], axis=0)
    pad = _N_PAD - rois_ext.shape[0]
    pad_rows = jnp.full((pad, 5), -1e9, dtype=jnp.float32)
    rois_p = jnp.concatenate([rois_ext, pad_rows], axis=0)
    roisT8 = jnp.concatenate(
        [rois_p.T, jnp.zeros((3, _N_PAD), jnp.float32)], axis=0)

    maxs, amax = pl.pallas_call(
        _iou_kern,
        out_shape=(
            jax.ShapeDtypeStruct((_NCHUNK, _CHUNK), jnp.float32),
            jax.ShapeDtypeStruct((_NCHUNK, _CHUNK), jnp.float32),
        ),
    )(roisT8, gt_boxes)

    rois16 = jnp.concatenate(
        [rois_p, jnp.zeros((_N_PAD, 3), jnp.float32),
         amax.reshape(_N_PAD, 1), jnp.zeros((_N_PAD, 7), jnp.float32)], axis=1)
    gt8 = jnp.concatenate([gt_boxes, jnp.zeros((G, 2), jnp.float32)], axis=1)

    kept, asg = _sc_call()(maxs.reshape(_N_PAD), rois16, gt8)

    outs = pl.pallas_call(
        _final_kern,
        out_shape=(
            jax.ShapeDtypeStruct((_BATCH, 5), jnp.float32),
            jax.ShapeDtypeStruct((_BATCH, 1), jnp.int32),
            jax.ShapeDtypeStruct((_BATCH, 1), jnp.int32),
            jax.ShapeDtypeStruct((_BATCH, 8), jnp.float32),
            jax.ShapeDtypeStruct((_BATCH, 8), jnp.float32),
            jax.ShapeDtypeStruct((_BATCH, 8), jnp.float32),
        ),
    )(kept, asg)
    rois, lab, pid, bt, biw, bow = outs
    return (rois, lab.reshape(_BATCH), pid.reshape(_BATCH), bt, biw, bow)


# speedup vs baseline: 1.0494x; 1.0494x over previous
import functools
import jax
import jax.numpy as jnp
from jax import lax
from jax.experimental import pallas as pl
from jax.experimental.pallas import tpu as pltpu, tpu_sc as plsc

_BG_PID = 5532.0
_BATCH = 128
_NUM_FG = 32
_FG_THRESH = 0.5
_BG_HI = 0.5
_BG_LO = 0.1
_N_PAD = 20480
_CHUNK = 2048
_NCHUNK = _N_PAD // _CHUNK
_STDS = (0.1, 0.1, 0.2, 0.2)

_NW = 16
_STRIP = _N_PAD // _NW
_NV = _STRIP // 16

_f32 = jnp.float32
_i32 = jnp.int32


def _iou_kern(roisT_ref, gt_ref, maxs_ref, amax_ref):
    qx1 = gt_ref[:, 0:1]
    qy1 = gt_ref[:, 1:2]
    qx2 = gt_ref[:, 2:3]
    qy2 = gt_ref[:, 3:4]
    qarea = (qx2 - qx1 + 1.0) * (qy2 - qy1 + 1.0)

    for c in range(_NCHUNK):
        sl = pl.ds(c * _CHUNK, _CHUNK)
        bx1 = roisT_ref[1:2, sl]
        by1 = roisT_ref[2:3, sl]
        bx2 = roisT_ref[3:4, sl]
        by2 = roisT_ref[4:5, sl]
        barea = (bx2 - bx1 + 1.0) * (by2 - by1 + 1.0)
        iw = jnp.maximum(jnp.minimum(bx2, qx2) - jnp.maximum(bx1, qx1) + 1.0, 0.0)
        ih = jnp.maximum(jnp.minimum(by2, qy2) - jnp.maximum(by1, qy1) + 1.0, 0.0)
        inter = iw * ih
        ua = barea + qarea - inter
        ov = inter / ua
        maxs_ref[c:c + 1, :] = jnp.max(ov, axis=0, keepdims=True)
        amax_ref[c:c + 1, :] = jnp.argmax(ov, axis=0, keepdims=True).astype(_f32)


def _sc_body(maxs_hbm, rois16_hbm, gt8_hbm, kept_hbm, asg_hbm,
             mx_v, keep_v, stage_v, scan_v, cnts_v, keeps_l, idx2,
             rows_v, asg_v, cnts_sh, keeps_sh, sem):
    wid = lax.axis_index("s")
    iot = lax.broadcasted_iota(_i32, (16,), 0)
    zero16 = jnp.zeros((16,), _i32)

    def prefix16(x):
        p = x
        for k in (1, 2, 4, 8):
            scan_v[...] = p
            sh = plsc.load_gather(scan_v, [jnp.maximum(iot - k, 0)])
            p = p + jnp.where(iot >= k, sh, 0)
        return p

    def tot(x):
        scan_v[...] = prefix16(x)
        return plsc.load_gather(scan_v, [jnp.full((16,), 15, _i32)])

    pltpu.sync_copy(maxs_hbm.at[pl.ds(wid * _STRIP, _STRIP)], mx_v)

    def p1(i, carry):
        cfg, cbg = carry
        v = mx_v[pl.ds(i * 16, 16)]
        fg = v >= _FG_THRESH
        bg = jnp.logical_and(v >= _BG_LO, v < _BG_HI)
        return cfg + jnp.where(fg, 1, 0), cbg + jnp.where(bg, 1, 0)

    cfg_l, cbg_l = lax.fori_loop(0, _NV, p1, (zero16, zero16))
    cfg = tot(cfg_l)
    cbg = tot(cbg_l)

    stage_v[...] = jnp.where(iot == 0, cfg, jnp.where(iot == 1, cbg, 0))
    pltpu.sync_copy(stage_v, cnts_sh.at[wid])
    plsc.subcore_barrier()
    pltpu.sync_copy(cnts_sh, cnts_v)

    c0f = plsc.load_gather(cnts_v, [iot, zero16])
    c0b = plsc.load_gather(cnts_v, [iot, jnp.ones((16,), _i32)])
    pref_fg = tot(jnp.where(iot < wid, c0f, 0))
    pref_bg = tot(jnp.where(iot < wid, c0b, 0))

    for j in range(8):
        keep_v[pl.ds(j * 16, 16)] = zero16

    def p3(i, carry):
        rfg, rbg = carry
        v = mx_v[pl.ds(i * 16, 16)]
        fg = v >= _FG_THRESH
        bg = jnp.logical_and(v >= _BG_LO, v < _BG_HI)
        pfg = prefix16(jnp.where(fg, 1, 0))
        pbg = prefix16(jnp.where(bg, 1, 0))
        t_fg = pref_fg + rfg + pfg - 1
        t_bg = pref_bg + rbg + pbg - 1
        selfg = jnp.logical_and(fg, t_fg < _NUM_FG)
        selbg = jnp.logical_and(bg, t_bg < _BATCH - _NUM_FG)
        slot = jnp.where(selfg, t_fg, jnp.where(selbg, t_bg + _NUM_FG, 0))
        msk = jnp.logical_or(selfg, selbg)
        gidx = wid * _STRIP + i * 16 + iot
        plsc.store_scatter(keep_v, [slot], gidx, mask=msk)
        scan_v[...] = pfg
        tfg = plsc.load_gather(scan_v, [jnp.full((16,), 15, _i32)])
        scan_v[...] = pbg
        tbg = plsc.load_gather(scan_v, [jnp.full((16,), 15, _i32)])
        return rfg + tfg, rbg + tbg

    lax.fori_loop(0, _NV, p3, (zero16, zero16))

    pltpu.sync_copy(keep_v, keeps_sh.at[wid])
    plsc.subcore_barrier()

    @pl.when(wid == 0)
    def _():
        pltpu.sync_copy(keeps_sh, keeps_l)
        for j in range(8):
            acc = zero16
            for r in range(_NW):
                acc = acc + keeps_l[r, pl.ds(j * 16, 16)]
            keep_v[pl.ds(j * 16, 16)] = acc
        pltpu.async_copy(rois16_hbm.at[keep_v], rows_v, sem).wait()
        for j in range(8):
            ak = plsc.load_gather(rows_v, [j * 16 + iot, jnp.full((16,), 8, _i32)])
            idx2[pl.ds(j * 16, 16)] = ak.astype(_i32)
        pltpu.async_copy(gt8_hbm.at[idx2], asg_v, sem).wait()
        pltpu.sync_copy(rows_v, kept_hbm)
        pltpu.sync_copy(asg_v, asg_hbm)


_sc_call = functools.partial(
    pl.kernel, _sc_body,
    out_type=(jax.ShapeDtypeStruct((_BATCH, 16), _f32),
              jax.ShapeDtypeStruct((_BATCH, 8), _f32)),
    mesh=plsc.VectorSubcoreMesh(core_axis_name="c", subcore_axis_name="s",
                                num_cores=1),
    compiler_params=pltpu.CompilerParams(needs_layout_passes=False,
                                         use_tc_tiling_on_sc=False),
    scratch_types=[
        pltpu.VMEM((_STRIP,), _f32),
        pltpu.VMEM((_BATCH,), _i32),
        pltpu.VMEM((16,), _i32),
        pltpu.VMEM((16,), _i32),
        pltpu.VMEM((_NW, 16), _i32),
        pltpu.VMEM((_NW, _BATCH), _i32),
        pltpu.VMEM((_BATCH,), _i32),
        pltpu.VMEM((_BATCH, 16), _f32),
        pltpu.VMEM((_BATCH, 8), _f32),
        pltpu.VMEM_SHARED((_NW, 16), _i32),
        pltpu.VMEM_SHARED((_NW, _BATCH), _i32),
        pltpu.SemaphoreType.DMA,
    ],
)


def _final_kern(kept_ref, asg_ref,
                rois_out, lab_out, pid_out, bt_out, biw_out, bow_out):
    kept = kept_ref[...]
    assigned = asg_ref[...]
    rx1 = kept[:, 1:2]
    ry1 = kept[:, 2:3]
    rx2 = kept[:, 3:4]
    ry2 = kept[:, 4:5]
    gx1 = assigned[:, 0:1]
    gy1 = assigned[:, 1:2]
    gx2 = assigned[:, 2:3]
    gy2 = assigned[:, 3:4]
    ew = rx2 - rx1 + 1.0
    eh = ry2 - ry1 + 1.0
    ecx = rx1 + 0.5 * ew
    ecy = ry1 + 0.5 * eh
    gw = gx2 - gx1 + 1.0
    gh = gy2 - gy1 + 1.0
    gcx = gx1 + 0.5 * gw
    gcy = gy1 + 0.5 * gh
    dx = (gcx - ecx) / ew / _STDS[0]
    dy = (gcy - ecy) / eh / _STDS[1]
    dw = jnp.log(gw / ew) / _STDS[2]
    dh = jnp.log(gh / eh) / _STDS[3]
    data = jnp.concatenate([dx, dy, dw, dh], axis=1)

    srow = lax.broadcasted_iota(_i32, (_BATCH, 1), 0).astype(_f32)
    isfg_slot = srow < float(_NUM_FG)
    labels = jnp.where(isfg_slot, assigned[:, 4:5], 0.0)
    pids = jnp.where(isfg_slot, assigned[:, 5:6], _BG_PID)
    fgw = (labels > 0.0).astype(_f32)
    clsr = jnp.round(labels)
    m0 = (clsr == 0.0).astype(_f32)
    m1 = (clsr == 1.0).astype(_f32)
    d = data * fgw
    ones4 = jnp.ones((_BATCH, 4), dtype=_f32)
    bt_out[...] = jnp.concatenate([d * m0, d * m1], axis=1)
    biw_out[...] = jnp.concatenate([ones4 * (fgw * m0), ones4 * (fgw * m1)], axis=1)
    bow_out[...] = jnp.concatenate([ones4 * (fgw * m0), ones4 * (fgw * m1)], axis=1)
    rois_out[...] = kept[:, 0:5]
    lab_out[...] = labels.astype(jnp.int32)
    pid_out[...] = pids.astype(jnp.int32)


@jax.jit
def kernel(all_rois, gt_boxes):
    G = gt_boxes.shape[0]
    gt_rois = jnp.concatenate(
        [jnp.zeros((G, 1), jnp.float32), gt_boxes[:, :4]], axis=1)
    rois_ext = jnp.concatenate([all_rois, gt_rois], axis=0)
    pad = _N_PAD - rois_ext.shape[0]
    pad_rows = jnp.full((pad, 5), -1e9, dtype=jnp.float32)
    rois_p = jnp.concatenate([rois_ext, pad_rows], axis=0)
    roisT8 = jnp.concatenate(
        [rois_p.T, jnp.zeros((3, _N_PAD), jnp.float32)], axis=0)

    maxs, amax = pl.pallas_call(
        _iou_kern,
        out_shape=(
            jax.ShapeDtypeStruct((_NCHUNK, _CHUNK), jnp.float32),
            jax.ShapeDtypeStruct((_NCHUNK, _CHUNK), jnp.float32),
        ),
    )(roisT8, gt_boxes)

    rois16 = jnp.concatenate(
        [rois_p, jnp.zeros((_N_PAD, 3), jnp.float32),
         amax.reshape(_N_PAD, 1), jnp.zeros((_N_PAD, 7), jnp.float32)], axis=1)
    gt8 = jnp.concatenate([gt_boxes, jnp.zeros((G, 2), jnp.float32)], axis=1)

    kept, asg = _sc_call()(maxs.reshape(_N_PAD), rois16, gt8)

    outs = pl.pallas_call(
        _final_kern,
        out_shape=(
            jax.ShapeDtypeStruct((_BATCH, 5), jnp.float32),
            jax.ShapeDtypeStruct((_BATCH, 1), jnp.int32),
            jax.ShapeDtypeStruct((_BATCH, 1), jnp.int32),
            jax.ShapeDtypeStruct((_BATCH, 8), jnp.float32),
            jax.ShapeDtypeStruct((_BATCH, 8), jnp.float32),
            jax.ShapeDtypeStruct((_BATCH, 8), jnp.float32),
        ),
    )(kept, asg)
    rois, lab, pid, bt, biw, bow = outs
    return (rois, lab.reshape(_BATCH), pid.reshape(_BATCH), bt, biw, bow)
